# baseline (speedup 1.0000x reference)
import jax
import jax.numpy as jnp
from jax import lax
from jax.experimental import pallas as pl
from jax.experimental.pallas import tpu as pltpu
from jax.experimental.pallas import tpu_sc as plsc


def _regroup(table_t, g, w):
    h, p = table_t.shape
    n_blocks = -(-p // (g * w))
    rows = n_blocks * w

    def _body(in_ref, out_ref):
        eye = (lax.broadcasted_iota(jnp.int32, (h, h), 0)
               == lax.broadcasted_iota(jnp.int32, (h, h), 1)).astype(jnp.float32)
        t = lax.dot_general(
            in_ref[...], eye, (((0,), (0,)), ((), ())),
            preferred_element_type=jnp.float32)
        for j in range(g):
            out_ref[:, j * h:(j + 1) * h] = t[j * w:(j + 1) * w, :]

    return pl.pallas_call(
        _body,
        grid=(n_blocks,),
        in_specs=[pl.BlockSpec((h, g * w), lambda i: (0, i))],
        out_specs=pl.BlockSpec((w, g * h), lambda i: (i, 0)),
        out_shape=jax.ShapeDtypeStruct((rows, g * h), jnp.float32),
    )(table_t)


def _sc_gather(table_g, idx, h, g, w):
    n_idx = idx.shape[0]
    gw = table_g.shape[1]
    info = plsc.get_sparse_core_info()
    nw = info.num_cores * info.num_subcores
    b_per_w = n_idx // nw
    nc = info.num_cores
    lanes = info.num_lanes
    w_shift = w.bit_length() - 1
    gw_shift = (g * w).bit_length() - 1

    def body(table_hbm, idx_hbm, out_hbm, idx_v, gidx_v, g_v, rows_v, sem):
        wid = lax.axis_index("s") * nc + lax.axis_index("c")
        base = wid * b_per_w
        pltpu.sync_copy(idx_hbm.at[pl.ds(base, b_per_w)], idx_v)
        for t in range(b_per_w // lanes):
            v = idx_v[pl.ds(t * lanes, lanes)]
            gi = (lax.shift_right_logical(v, gw_shift) * w) + (v & (w - 1))
            gidx_v[pl.ds(t * lanes, lanes)] = gi
        pltpu.async_copy(table_hbm.at[gidx_v], g_v, sem).wait()
        for t in range(b_per_w // lanes):
            rows16 = lax.iota(jnp.int32, lanes) + t * lanes
            v = idx_v[pl.ds(t * lanes, lanes)]
            off16 = (lax.shift_right_logical(v, w_shift) & (g - 1)) * h
            for c in range(h):
                vals = plsc.load_gather(g_v, [rows16, off16 + c])
                plsc.store_scatter(
                    rows_v, [rows16, jnp.full((lanes,), c, jnp.int32)], vals)
        pltpu.sync_copy(rows_v, out_hbm.at[pl.ds(base, b_per_w)])

    gather = pl.kernel(
        body,
        out_type=jax.ShapeDtypeStruct((n_idx, h), table_g.dtype),
        mesh=plsc.VectorSubcoreMesh(core_axis_name="c", subcore_axis_name="s"),
        scratch_types=[
            pltpu.VMEM((b_per_w,), jnp.int32),
            pltpu.VMEM((b_per_w,), jnp.int32),
            pltpu.VMEM((b_per_w, gw), table_g.dtype),
            pltpu.VMEM((b_per_w, h), table_g.dtype),
            pltpu.SemaphoreType.DMA,
        ],
        compiler_params=pltpu.CompilerParams(needs_layout_passes=False),
    )
    return gather(table_g, idx)


def _mlp_body(e2_ref, w1_ref, b1_ref, w2t_ref, b2_ref, out_ref, h_ref):
    batch = h_ref.shape[0]
    hid = h_ref.shape[1]

    @pl.when(pl.program_id(0) == 0)
    def _():
        h = lax.dot_general(
            e2_ref[:batch, :], w1_ref[:, :hid], (((1,), (1,)), ((), ())),
            preferred_element_type=jnp.float32)
        h = h + lax.dot_general(
            e2_ref[batch:, :], w1_ref[:, hid:], (((1,), (1,)), ((), ())),
            preferred_element_type=jnp.float32)
        h_ref[...] = jnp.maximum(h + b1_ref[...], 0.0)

    bias = lax.dot_general(
        b2_ref[...], jnp.ones((1, batch), jnp.float32), (((0,), (0,)), ((), ())),
        preferred_element_type=jnp.float32)
    out_ref[...] = lax.dot_general(
        w2t_ref[...], h_ref[...], (((0,), (1,)), ((), ())),
        preferred_element_type=jnp.float32) + bias


def kernel(x, embed_table, w1, b1, w2, b2):
    batch, ids_per_row = x.shape
    p, hidden = embed_table.shape
    e_dim = ids_per_row * hidden

    idx = x.T.reshape(-1).astype(jnp.int32)
    g = 128 // hidden
    w = 4096
    table_g = _regroup(embed_table.T, g, w)
    e2 = _sc_gather(table_g, idx, hidden, g, w)

    bp = 5120
    np_blocks = (p + bp - 1) // bp

    out_t = pl.pallas_call(
        _mlp_body,
        grid=(np_blocks,),
        in_specs=[
            pl.BlockSpec((ids_per_row * batch, hidden), lambda i: (0, 0)),
            pl.BlockSpec((hidden, e_dim), lambda i: (0, 0)),
            pl.BlockSpec((1, hidden), lambda i: (0, 0)),
            pl.BlockSpec((hidden, bp), lambda i: (0, i)),
            pl.BlockSpec((1, bp), lambda i: (0, i)),
        ],
        out_specs=pl.BlockSpec((bp, batch), lambda i: (i, 0)),
        out_shape=jax.ShapeDtypeStruct((p, batch), jnp.float32),
        scratch_shapes=[pltpu.VMEM((batch, hidden), jnp.float32)],
    )(e2, w1, b1[None, :], w2.T, b2[None, :])
    return out_t.T


# speedup vs baseline: 1.0105x; 1.0105x over previous
import jax
import jax.numpy as jnp
from jax import lax
from jax.experimental import pallas as pl
from jax.experimental.pallas import tpu as pltpu
from jax.experimental.pallas import tpu_sc as plsc


def _regroup(table_t, g, w):
    h, p = table_t.shape
    n_blocks = -(-p // (g * w))
    rows = n_blocks * w

    def _body(in_ref, out_ref):
        eye = (lax.broadcasted_iota(jnp.int32, (h, h), 0)
               == lax.broadcasted_iota(jnp.int32, (h, h), 1)).astype(jnp.float32)
        t = lax.dot_general(
            in_ref[...], eye, (((0,), (0,)), ((), ())),
            preferred_element_type=jnp.float32)
        for j in range(g):
            out_ref[:, j * h:(j + 1) * h] = t[j * w:(j + 1) * w, :]

    return pl.pallas_call(
        _body,
        grid=(n_blocks,),
        in_specs=[pl.BlockSpec((h, g * w), lambda i: (0, i))],
        out_specs=pl.BlockSpec((w, g * h), lambda i: (i, 0)),
        out_shape=jax.ShapeDtypeStruct((rows, g * h), jnp.float32),
    )(table_t)


def _sc_gather(table_g, idx, h, g, w):
    n_idx = idx.shape[0]
    gw = table_g.shape[1]
    info = plsc.get_sparse_core_info()
    nw = info.num_cores * info.num_subcores
    b_per_w = n_idx // nw
    nc = info.num_cores
    lanes = info.num_lanes
    w_shift = w.bit_length() - 1
    gw_shift = (g * w).bit_length() - 1

    def body(table_hbm, idx_hbm, out_hbm, idx_v, gidx_v, g_v, rows_v, sem):
        wid = lax.axis_index("s") * nc + lax.axis_index("c")
        base = wid * b_per_w
        pltpu.sync_copy(idx_hbm.at[pl.ds(base, b_per_w)], idx_v)
        for t in range(b_per_w // lanes):
            v = idx_v[pl.ds(t * lanes, lanes)]
            gi = (lax.shift_right_logical(v, gw_shift) * w) + (v & (w - 1))
            gidx_v[pl.ds(t * lanes, lanes)] = gi
        pltpu.async_copy(table_hbm.at[gidx_v], g_v, sem).wait()
        for t in range(b_per_w // lanes):
            rows16 = lax.iota(jnp.int32, lanes) + t * lanes
            v = idx_v[pl.ds(t * lanes, lanes)]
            off16 = (lax.shift_right_logical(v, w_shift) & (g - 1)) * h
            for c in range(h):
                vals = plsc.load_gather(g_v, [rows16, off16 + c])
                plsc.store_scatter(
                    rows_v, [rows16, jnp.full((lanes,), c, jnp.int32)], vals)
        pltpu.sync_copy(rows_v, out_hbm.at[pl.ds(base, b_per_w)])

    gather = pl.kernel(
        body,
        out_type=jax.ShapeDtypeStruct((n_idx, h), table_g.dtype),
        mesh=plsc.VectorSubcoreMesh(core_axis_name="c", subcore_axis_name="s"),
        scratch_types=[
            pltpu.VMEM((b_per_w,), jnp.int32),
            pltpu.VMEM((b_per_w,), jnp.int32),
            pltpu.VMEM((b_per_w, gw), table_g.dtype),
            pltpu.VMEM((b_per_w, h), table_g.dtype),
            pltpu.SemaphoreType.DMA,
        ],
        compiler_params=pltpu.CompilerParams(needs_layout_passes=False),
    )
    return gather(table_g, idx)


def _mlp_body(e2_ref, w1_ref, b1_ref, w2t_ref, b2_ref, out_ref, h_ref):
    batch = h_ref.shape[0]
    hid = h_ref.shape[1]

    @pl.when(pl.program_id(0) == 0)
    def _():
        h = lax.dot_general(
            e2_ref[:batch, :], w1_ref[:, :hid], (((1,), (1,)), ((), ())),
            preferred_element_type=jnp.float32)
        h = h + lax.dot_general(
            e2_ref[batch:, :], w1_ref[:, hid:], (((1,), (1,)), ((), ())),
            preferred_element_type=jnp.float32)
        h_ref[...] = jnp.maximum(h + b1_ref[...], 0.0)

    bias = lax.dot_general(
        b2_ref[...], jnp.ones((1, batch), jnp.float32), (((0,), (0,)), ((), ())),
        preferred_element_type=jnp.float32)
    out_ref[...] = lax.dot_general(
        w2t_ref[...], h_ref[...], (((0,), (1,)), ((), ())),
        preferred_element_type=jnp.float32) + bias


def kernel(x, embed_table, w1, b1, w2, b2):
    batch, ids_per_row = x.shape
    p, hidden = embed_table.shape
    e_dim = ids_per_row * hidden

    idx = x.T.reshape(-1).astype(jnp.int32)
    g = 128 // hidden
    w = 2048
    table_g = _regroup(embed_table.T, g, w)
    e2 = _sc_gather(table_g, idx, hidden, g, w)

    bp = 4096
    np_blocks = (p + bp - 1) // bp

    out_t = pl.pallas_call(
        _mlp_body,
        grid=(np_blocks,),
        in_specs=[
            pl.BlockSpec((ids_per_row * batch, hidden), lambda i: (0, 0)),
            pl.BlockSpec((hidden, e_dim), lambda i: (0, 0)),
            pl.BlockSpec((1, hidden), lambda i: (0, 0)),
            pl.BlockSpec((hidden, bp), lambda i: (0, i)),
            pl.BlockSpec((1, bp), lambda i: (0, i)),
        ],
        out_specs=pl.BlockSpec((bp, batch), lambda i: (i, 0)),
        out_shape=jax.ShapeDtypeStruct((p, batch), jnp.float32),
        scratch_shapes=[pltpu.VMEM((batch, hidden), jnp.float32)],
    )(e2, w1, b1[None, :], w2.T, b2[None, :])
    return out_t.T
